# Initial kernel scaffold; baseline (speedup 1.0000x reference)
import jax
import jax.numpy as jnp
from jax.experimental import pallas as pl


def kernel(x):
    raise NotImplementedError("write your pallas kernel here")


# speedup vs baseline: 16.1702x; 16.1702x over previous
import functools

import jax
import jax.numpy as jnp
from jax.experimental import pallas as pl

_K = 50
_ROWS_PER_BLOCK = 16


def _mask_kernel(x_ref, o_ref):
    x = x_ref[...]
    u = jax.lax.bitcast_convert_type(x, jnp.uint32)
    neg = (u >> 31) == 1
    ukey = jnp.where(neg, ~u, u | jnp.uint32(0x80000000))

    rows = x.shape[0]

    def body(it, p):
        i = 31 - it
        c = p | (jnp.uint32(1) << i)
        ge = (ukey >= c).astype(jnp.int32)
        cnt = jnp.sum(ge, axis=1, keepdims=True)
        return jnp.where(cnt >= _K, c, p)

    p0 = jnp.zeros((rows, 1), dtype=jnp.uint32)
    kv = jax.lax.fori_loop(0, 32, body, p0)

    topbit = (kv >> 31) == 1
    u_orig = jnp.where(topbit, kv & jnp.uint32(0x7FFFFFFF), ~kv)
    tv = jax.lax.bitcast_convert_type(u_orig, jnp.float32)
    o_ref[...] = (x >= tv).astype(jnp.float32)


@jax.jit
def kernel(x):
    m, n = x.shape
    r = _ROWS_PER_BLOCK
    grid = (m // r,)
    return pl.pallas_call(
        _mask_kernel,
        out_shape=jax.ShapeDtypeStruct((m, n), jnp.float32),
        grid=grid,
        in_specs=[pl.BlockSpec((r, n), lambda i: (i, 0))],
        out_specs=pl.BlockSpec((r, n), lambda i: (i, 0)),
    )(x)


# speedup vs baseline: 27.2878x; 1.6875x over previous
import jax
import jax.numpy as jnp
from jax.experimental import pallas as pl

_K = 50
_ROWS_PER_BLOCK = 16


def _count_cmp(v_s, c_s, strict):
    n = v_s.shape[1]
    w = n // 32
    one = jnp.bfloat16(1)
    zero = jnp.bfloat16(0)
    t = None
    for j in range(32):
        sl = v_s[:, j * w : (j + 1) * w]
        m = (sl > c_s) if strict else (sl >= c_s)
        part = jnp.where(m, one, zero)
        t = part if t is None else t + part
    while w > 128:
        half = w // 2
        t = t[:, :half] + t[:, half:]
        w = half
    f = t.astype(jnp.float32)
    return jnp.sum(f, axis=1, keepdims=True)


def _to_s16(c):
    return (c ^ jnp.int32(0x8000)).astype(jnp.int16)


def _search16(v_s, target):
    rows = v_s.shape[0]
    p = jnp.zeros((rows, 1), dtype=jnp.int32)
    for i in range(15, -1, -1):
        c = p | jnp.int32(1 << i)
        cnt = _count_cmp(v_s, _to_s16(c), strict=False)
        p = jnp.where(cnt >= target, c, p)
    return p


def _mask_kernel(x_ref, o_ref):
    x = x_ref[...]
    u = jax.lax.bitcast_convert_type(x, jnp.uint32)
    neg = (u >> 31) == 1
    ukey = jnp.where(neg, ~u, u | jnp.uint32(0x80000000))
    ikey = jax.lax.bitcast_convert_type(ukey, jnp.int32)

    hi_s = ((ikey >> 16) ^ jnp.int32(0x8000)).astype(jnp.int16)
    lo_s = (ikey ^ jnp.int32(0x8000)).astype(jnp.int16)
    k = jnp.float32(_K)

    p_hi = _search16(hi_s, k)
    p_hi_s = _to_s16(p_hi)

    cnt_gt = _count_cmp(hi_s, p_hi_s, strict=True)

    z_s = jnp.where(hi_s == p_hi_s, lo_s, jnp.int16(-32768))

    p_lo = _search16(z_s, k - cnt_gt)

    kv = jax.lax.bitcast_convert_type((p_hi << 16) | p_lo, jnp.uint32)

    topbit = (kv >> 31) == 1
    u_orig = jnp.where(topbit, kv & jnp.uint32(0x7FFFFFFF), ~kv)
    tv = jax.lax.bitcast_convert_type(u_orig, jnp.float32)
    o_ref[...] = (x >= tv).astype(jnp.float32)


@jax.jit
def kernel(x):
    m, n = x.shape
    r = _ROWS_PER_BLOCK
    grid = (m // r,)
    return pl.pallas_call(
        _mask_kernel,
        out_shape=jax.ShapeDtypeStruct((m, n), jnp.float32),
        grid=grid,
        in_specs=[pl.BlockSpec((r, n), lambda i: (i, 0))],
        out_specs=pl.BlockSpec((r, n), lambda i: (i, 0)),
    )(x)
